# baseline (speedup 1.0000x reference)
import functools

import jax
import jax.numpy as jnp
from jax import lax
from jax.experimental import pallas as pl
from jax.experimental.pallas import tpu as pltpu
from jax.experimental.pallas import tpu_sc as plsc


N_NODES = 100000
N_EDGES = 3200000
NSC = 2
NSUB = 16
RNG = 6256
NPAD = RNG * NSUB
EPC = N_EDGES // NSC
W = 2000
NW = EPC // W
VPW = W // 16


def _sc_agg_body(eiflat_hbm, aflat_hbm, z1_hbm, negf_hbm, ones_hbm,
                 sums_out, cnts_out, maxs_out,
                 col0, col1, fattr0, fattr1, onesbuf, zbuf, stg, acc,
                 sp0, sp1, sp2, sp3, spc,
                 sem_c0, sem_c1, sem_a0, sem_a1, sem_add):
    c = lax.axis_index("c")
    s = lax.axis_index("s")
    lo = s * RNG
    ebase = c * EPC
    splanes = (sp0, sp1, sp2, sp3)

    pltpu.sync_copy(z1_hbm, zbuf)
    for sp in (sp0, sp1, sp2, sp3, spc):
        for i in range(RNG // W):
            pltpu.sync_copy(zbuf, sp.at[pl.ds(lo + i * W, W)])
        rem = RNG % W
        pltpu.sync_copy(zbuf.at[pl.ds(0, rem)],
                        sp.at[pl.ds(lo + (RNG // W) * W, rem)])
    pltpu.sync_copy(negf_hbm, acc)
    pltpu.sync_copy(ones_hbm, onesbuf)
    plsc.subcore_barrier()

    iota16 = lax.iota(jnp.int32, 16)

    cbufs = (col0, col1)
    abufs = (fattr0, fattr1)
    csems = (sem_c0, sem_c1)
    asems = (sem_a0, sem_a1)

    def start(w, b):
        off = ebase + w * W
        pltpu.async_copy(eiflat_hbm.at[pl.ds(N_EDGES + off, W)], cbufs[b],
                         csems[b])
        for j in range(4):
            pltpu.async_copy(aflat_hbm.at[pl.ds(j * N_EDGES + off, W)],
                             abufs[b].at[pl.ds(j * W, W)], asems[b])

    def wait(w, b):
        off = ebase + w * W
        pltpu.make_async_copy(eiflat_hbm.at[pl.ds(N_EDGES + off, W)],
                              cbufs[b], csems[b]).wait()
        for j in range(4):
            pltpu.make_async_copy(aflat_hbm.at[pl.ds(j * N_EDGES + off, W)],
                                  abufs[b].at[pl.ds(j * W, W)],
                                  asems[b]).wait()

    start(0, 0)
    start(1, 1)

    def window_body(i, _):
        for b in (0, 1):
            w = 2 * i + b
            colb = cbufs[b]
            attrb = abufs[b]
            wait(w, b)

            mine = lax.rem(w, NSUB) == s

            @pl.when(mine)
            def _():
                for j in range(4):
                    pltpu.async_copy(attrb.at[pl.ds(j * W, W)],
                                     splanes[j].at[colb], sem_add, add=True)
                pltpu.async_copy(onesbuf, spc.at[colb], sem_add, add=True)

            def vreg_body(k, _):
                colv = colb[pl.ds(k * 16, 16)]
                loc = colv - lo
                m = (loc >= 0) & (loc < RNG)
                locs = jnp.where(m, loc, 0)
                trash = 4 * RNG + iota16

                @pl.when(jnp.any(m))
                def _():
                    avs = [attrb[pl.ds(j * W + k * 16, 16)]
                           for j in range(4)]

                    def update(mm):
                        for j in range(4):
                            idxj = jnp.where(mm, j * RNG + locs, trash)
                            cur = plsc.load_gather(acc, [idxj])
                            plsc.store_scatter(acc, [idxj],
                                               jnp.maximum(cur, avs[j]))

                    def verify(mm):
                        bad = jnp.zeros((16,), jnp.bool_)
                        for j in range(4):
                            idxj = jnp.where(mm, j * RNG + locs, trash)
                            cur2 = plsc.load_gather(acc, [idxj])
                            bad = bad | (mm & (cur2 < avs[j]))
                        return bad

                    update(m)

                    @pl.when(jnp.sum(m.astype(jnp.int32)) >= 2)
                    def _():
                        def body(mm):
                            update(mm)
                            return verify(mm)

                        lax.while_loop(lambda mm: jnp.any(mm), body,
                                       verify(m))
                return 0

            lax.fori_loop(0, VPW, vreg_body, 0)

            @pl.when(mine)
            def _():
                for j in range(4):
                    pltpu.make_async_copy(attrb.at[pl.ds(j * W, W)],
                                          splanes[j].at[colb],
                                          sem_add).wait()
                pltpu.make_async_copy(onesbuf, spc.at[colb], sem_add).wait()

            @pl.when(w + 2 < NW)
            def _():
                start(w + 2, b)
        return 0

    lax.fori_loop(0, NW // 2, window_body, 0)
    plsc.subcore_barrier()

    for j in range(4):
        pltpu.sync_copy(splanes[j].at[pl.ds(lo, RNG)], stg)
        pltpu.sync_copy(stg, sums_out.at[pl.ds((c * 4 + j) * NPAD + lo, RNG)])
    pltpu.sync_copy(spc.at[pl.ds(lo, RNG)], stg)
    pltpu.sync_copy(stg, cnts_out.at[pl.ds(c * NPAD + lo, RNG)])
    for j in range(4):
        pltpu.sync_copy(acc.at[pl.ds(j * RNG, RNG)],
                        maxs_out.at[pl.ds((c * 4 + j) * NPAD + lo, RNG)])


def _run_sc_agg(eiflat, aflat):
    mesh = plsc.VectorSubcoreMesh(core_axis_name="c", subcore_axis_name="s")
    z1 = jnp.zeros((W,), jnp.float32)
    negf = jnp.full((4 * RNG + 16,), -jnp.inf, jnp.float32)
    ones = jnp.ones((W,), jnp.float32)
    kfn = pl.kernel(
        _sc_agg_body,
        out_type=(
            jax.ShapeDtypeStruct((NSC * 4 * NPAD,), jnp.float32),
            jax.ShapeDtypeStruct((NSC * NPAD,), jnp.float32),
            jax.ShapeDtypeStruct((NSC * 4 * NPAD,), jnp.float32),
        ),
        mesh=mesh,
        scratch_types=[
            pltpu.VMEM((W,), jnp.int32),
            pltpu.VMEM((W,), jnp.int32),
            pltpu.VMEM((4 * W,), jnp.float32),
            pltpu.VMEM((4 * W,), jnp.float32),
            pltpu.VMEM((W,), jnp.float32),
            pltpu.VMEM((W,), jnp.float32),
            pltpu.VMEM((RNG,), jnp.float32),
            pltpu.VMEM((4 * RNG + 16,), jnp.float32),
            pltpu.VMEM_SHARED((NPAD,), jnp.float32),
            pltpu.VMEM_SHARED((NPAD,), jnp.float32),
            pltpu.VMEM_SHARED((NPAD,), jnp.float32),
            pltpu.VMEM_SHARED((NPAD,), jnp.float32),
            pltpu.VMEM_SHARED((NPAD,), jnp.float32),
            pltpu.SemaphoreType.DMA,
            pltpu.SemaphoreType.DMA,
            pltpu.SemaphoreType.DMA,
            pltpu.SemaphoreType.DMA,
            pltpu.SemaphoreType.DMA,
        ],
        compiler_params=pltpu.CompilerParams(needs_layout_passes=False,
                                             use_tc_tiling_on_sc=False),
    )
    return kfn(eiflat, aflat, z1, negf, ones)


def _mlp_body(x_ref, batch_ref, u_ref, w1x_ref, w1a_ref, w1u_ref, b1_ref,
              w2_ref, b2_ref, *rest):
    (s00, s01, s02, s03, s10, s11, s12, s13, c0, c1,
     m00, m01, m02, m03, m10, m11, m12, m13, out_ref) = rest
    s0p = (s00, s01, s02, s03)
    s1p = (s10, s11, s12, s13)
    m0p = (m00, m01, m02, m03)
    m1p = (m10, m11, m12, m13)

    cnt = (c0[0, 0, :] + c1[0, 0, :])[:, None]
    sj = [(s0p[j][0, 0, :] + s1p[j][0, 0, :])[:, None] for j in range(4)]
    mj = [jnp.maximum(m0p[j][0, 0, :], m1p[j][0, 0, :])[:, None]
          for j in range(4)]
    pos = cnt > 0
    inv = 1.0 / jnp.maximum(cnt, 1.0)
    agg = jnp.concatenate(
        sj + [jnp.where(pos, m, 0.0) for m in mj] + [s * inv for s in sj],
        axis=1)

    b = batch_ref[0, 0, :]
    oneh = (b[:, None] == lax.broadcasted_iota(jnp.int32, (1, 16), 1)
            ).astype(jnp.float32)
    uw = jnp.dot(u_ref[...], w1u_ref[...], preferred_element_type=jnp.float32)

    h = jnp.dot(x_ref[...], w1x_ref[...], preferred_element_type=jnp.float32)
    h += jnp.dot(agg, w1a_ref[...], preferred_element_type=jnp.float32)
    h += jnp.dot(oneh, uw, preferred_element_type=jnp.float32)
    h = jnp.maximum(h + b1_ref[...], 0.0)
    out_ref[...] = jnp.dot(h, w2_ref[...],
                           preferred_element_type=jnp.float32) + b2_ref[...]


def _run_mlp(x, sums_f, cnts_f, maxs_f, batch, u, W1, b1, W2, b2):
    n, node_in = x.shape
    blk = 1000
    grid = n // blk
    w1x = W1[0:node_in]
    w1a = W1[node_in:node_in + 12]
    w1u = W1[node_in + 12:]
    batch3 = batch.astype(jnp.int32).reshape(grid, 1, blk)

    def plane(f, off):
        return f[off:off + n].reshape(grid, 1, blk)

    splanes = [plane(sums_f, (c * 4 + j) * NPAD)
               for c in range(NSC) for j in range(4)]
    cplanes = [plane(cnts_f, c * NPAD) for c in range(NSC)]
    mplanes = [plane(maxs_f, (c * 4 + j) * NPAD)
               for c in range(NSC) for j in range(4)]

    full = lambda shape: pl.BlockSpec(shape, lambda i: (0,) * len(shape))
    p3 = pl.BlockSpec((1, 1, blk), lambda i: (i, 0, 0))
    return pl.pallas_call(
        _mlp_body,
        grid=(grid,),
        in_specs=[
            pl.BlockSpec((blk, node_in), lambda i: (i, 0)),
            p3,
            full(u.shape),
            full(w1x.shape),
            full(w1a.shape),
            full(w1u.shape),
            full((1, 128)),
            full(W2.shape),
            full((1, 128)),
        ] + [p3] * 18,
        out_specs=pl.BlockSpec((blk, 128), lambda i: (i, 0)),
        out_shape=jax.ShapeDtypeStruct((n, 128), jnp.float32),
        compiler_params=pltpu.CompilerParams(
            dimension_semantics=("arbitrary",),
        ),
    )(x, batch3, u, w1x, w1a, w1u, b1.reshape(1, 128), W2,
      b2.reshape(1, 128), *splanes, *cplanes, *mplanes)


def kernel(x, edge_index, edge_attr, u, batch, W1, b1, W2, b2):
    n = x.shape[0]
    eiflat = edge_index.astype(jnp.int32).reshape(-1)
    aflat = edge_attr.T.reshape(-1)
    eiflat, aflat = lax.optimization_barrier((eiflat, aflat))
    sums_f, cnts_f, maxs_f = _run_sc_agg(eiflat, aflat)
    return _run_mlp(x, sums_f, cnts_f, maxs_f, batch, u, W1, b1, W2, b2)


# speedup vs baseline: 1.3217x; 1.3217x over previous
import functools

import jax
import jax.numpy as jnp
from jax import lax
from jax.experimental import pallas as pl
from jax.experimental.pallas import tpu as pltpu
from jax.experimental.pallas import tpu_sc as plsc


N_NODES = 100000
N_EDGES = 3200000
NSC = 2
NSUB = 16
RNG = 6256
NPAD = RNG * NSUB
EPC = N_EDGES // NSC
W = 2000
NW = EPC // W
VPW = W // 16


def _sc_agg_body(eiflat_hbm, aflat_hbm, z1_hbm, negf_hbm, ones_hbm,
                 sums_out, cnts_out, maxs_out,
                 col0, col1, fattr0, fattr1, onesbuf, zbuf, stg, acc,
                 sp0, sp1, sp2, sp3, spc,
                 sem_c0, sem_c1, sem_a0, sem_a1, sem_add):
    c = lax.axis_index("c")
    s = lax.axis_index("s")
    lo = s * RNG
    ebase = c * EPC
    splanes = (sp0, sp1, sp2, sp3)

    pltpu.sync_copy(z1_hbm, zbuf)
    for sp in (sp0, sp1, sp2, sp3, spc):
        for i in range(RNG // W):
            pltpu.sync_copy(zbuf, sp.at[pl.ds(lo + i * W, W)])
        rem = RNG % W
        pltpu.sync_copy(zbuf.at[pl.ds(0, rem)],
                        sp.at[pl.ds(lo + (RNG // W) * W, rem)])
    pltpu.sync_copy(negf_hbm, acc)
    pltpu.sync_copy(ones_hbm, onesbuf)
    plsc.subcore_barrier()

    iota16 = lax.iota(jnp.int32, 16)

    cbufs = (col0, col1)
    abufs = (fattr0, fattr1)
    csems = (sem_c0, sem_c1)
    asems = (sem_a0, sem_a1)

    def start(w, b):
        off = ebase + w * W
        pltpu.async_copy(eiflat_hbm.at[pl.ds(N_EDGES + off, W)], cbufs[b],
                         csems[b])
        for j in range(4):
            pltpu.async_copy(aflat_hbm.at[pl.ds(j * N_EDGES + off, W)],
                             abufs[b].at[pl.ds(j * W, W)], asems[b])

    def wait(w, b):
        off = ebase + w * W
        pltpu.make_async_copy(eiflat_hbm.at[pl.ds(N_EDGES + off, W)],
                              cbufs[b], csems[b]).wait()
        for j in range(4):
            pltpu.make_async_copy(aflat_hbm.at[pl.ds(j * N_EDGES + off, W)],
                                  abufs[b].at[pl.ds(j * W, W)],
                                  asems[b]).wait()

    start(0, 0)
    start(1, 1)

    def window_body(i, _):
        for b in (0, 1):
            w = 2 * i + b
            colb = cbufs[b]
            attrb = abufs[b]
            wait(w, b)

            mine = lax.rem(w, NSUB) == s

            @pl.when(mine)
            def _():
                for j in range(4):
                    pltpu.async_copy(attrb.at[pl.ds(j * W, W)],
                                     splanes[j].at[colb], sem_add, add=True)
                pltpu.async_copy(onesbuf, spc.at[colb], sem_add, add=True)

            def vreg_body(k, _):
                colv = colb[pl.ds(k * 16, 16)]
                loc = colv - lo
                m = (loc >= 0) & (loc < RNG)
                locs = jnp.where(m, loc, 0)
                avs = [attrb[pl.ds(j * W + k * 16, 16)] for j in range(4)]
                trash = 4 * RNG + iota16

                def rmw(mm):
                    bad = jnp.zeros((16,), jnp.bool_)
                    for j in range(4):
                        idxj = jnp.where(mm, j * RNG + locs, trash)
                        cur = plsc.load_gather(acc, [idxj])
                        new = jnp.maximum(cur, avs[j])
                        plsc.store_scatter(acc, [idxj], new)
                        cur2 = plsc.load_gather(acc, [idxj])
                        bad = bad | (mm & (cur2 < avs[j]))
                    return bad

                bad = rmw(m)
                lax.while_loop(lambda mm: jnp.any(mm), rmw, bad)
                return 0

            lax.fori_loop(0, VPW, vreg_body, 0)

            @pl.when(mine)
            def _():
                for j in range(4):
                    pltpu.make_async_copy(attrb.at[pl.ds(j * W, W)],
                                          splanes[j].at[colb],
                                          sem_add).wait()
                pltpu.make_async_copy(onesbuf, spc.at[colb], sem_add).wait()

            @pl.when(w + 2 < NW)
            def _():
                start(w + 2, b)
        return 0

    lax.fori_loop(0, NW // 2, window_body, 0)
    plsc.subcore_barrier()

    for j in range(4):
        pltpu.sync_copy(splanes[j].at[pl.ds(lo, RNG)], stg)
        pltpu.sync_copy(stg, sums_out.at[pl.ds((c * 4 + j) * NPAD + lo, RNG)])
    pltpu.sync_copy(spc.at[pl.ds(lo, RNG)], stg)
    pltpu.sync_copy(stg, cnts_out.at[pl.ds(c * NPAD + lo, RNG)])
    for j in range(4):
        pltpu.sync_copy(acc.at[pl.ds(j * RNG, RNG)],
                        maxs_out.at[pl.ds((c * 4 + j) * NPAD + lo, RNG)])


def _run_sc_agg(eiflat, aflat):
    mesh = plsc.VectorSubcoreMesh(core_axis_name="c", subcore_axis_name="s")
    z1 = jnp.zeros((W,), jnp.float32)
    negf = jnp.full((4 * RNG + 16,), -jnp.inf, jnp.float32)
    ones = jnp.ones((W,), jnp.float32)
    kfn = pl.kernel(
        _sc_agg_body,
        out_type=(
            jax.ShapeDtypeStruct((NSC * 4 * NPAD,), jnp.float32),
            jax.ShapeDtypeStruct((NSC * NPAD,), jnp.float32),
            jax.ShapeDtypeStruct((NSC * 4 * NPAD,), jnp.float32),
        ),
        mesh=mesh,
        scratch_types=[
            pltpu.VMEM((W,), jnp.int32),
            pltpu.VMEM((W,), jnp.int32),
            pltpu.VMEM((4 * W,), jnp.float32),
            pltpu.VMEM((4 * W,), jnp.float32),
            pltpu.VMEM((W,), jnp.float32),
            pltpu.VMEM((W,), jnp.float32),
            pltpu.VMEM((RNG,), jnp.float32),
            pltpu.VMEM((4 * RNG + 16,), jnp.float32),
            pltpu.VMEM_SHARED((NPAD,), jnp.float32),
            pltpu.VMEM_SHARED((NPAD,), jnp.float32),
            pltpu.VMEM_SHARED((NPAD,), jnp.float32),
            pltpu.VMEM_SHARED((NPAD,), jnp.float32),
            pltpu.VMEM_SHARED((NPAD,), jnp.float32),
            pltpu.SemaphoreType.DMA,
            pltpu.SemaphoreType.DMA,
            pltpu.SemaphoreType.DMA,
            pltpu.SemaphoreType.DMA,
            pltpu.SemaphoreType.DMA,
        ],
        compiler_params=pltpu.CompilerParams(needs_layout_passes=False,
                                             use_tc_tiling_on_sc=False),
    )
    return kfn(eiflat, aflat, z1, negf, ones)


def _mlp_body(x_ref, batch_ref, u_ref, w1x_ref, w1a_ref, w1u_ref, b1_ref,
              w2_ref, b2_ref, *rest):
    (s00, s01, s02, s03, s10, s11, s12, s13, c0, c1,
     m00, m01, m02, m03, m10, m11, m12, m13, out_ref) = rest
    s0p = (s00, s01, s02, s03)
    s1p = (s10, s11, s12, s13)
    m0p = (m00, m01, m02, m03)
    m1p = (m10, m11, m12, m13)

    cnt = (c0[0, 0, :] + c1[0, 0, :])[:, None]
    sj = [(s0p[j][0, 0, :] + s1p[j][0, 0, :])[:, None] for j in range(4)]
    mj = [jnp.maximum(m0p[j][0, 0, :], m1p[j][0, 0, :])[:, None]
          for j in range(4)]
    pos = cnt > 0
    inv = 1.0 / jnp.maximum(cnt, 1.0)
    agg = jnp.concatenate(
        sj + [jnp.where(pos, m, 0.0) for m in mj] + [s * inv for s in sj],
        axis=1)

    b = batch_ref[0, 0, :]
    oneh = (b[:, None] == lax.broadcasted_iota(jnp.int32, (1, 16), 1)
            ).astype(jnp.float32)
    uw = jnp.dot(u_ref[...], w1u_ref[...], preferred_element_type=jnp.float32)

    h = jnp.dot(x_ref[...], w1x_ref[...], preferred_element_type=jnp.float32)
    h += jnp.dot(agg, w1a_ref[...], preferred_element_type=jnp.float32)
    h += jnp.dot(oneh, uw, preferred_element_type=jnp.float32)
    h = jnp.maximum(h + b1_ref[...], 0.0)
    out_ref[...] = jnp.dot(h, w2_ref[...],
                           preferred_element_type=jnp.float32) + b2_ref[...]


def _run_mlp(x, sums_f, cnts_f, maxs_f, batch, u, W1, b1, W2, b2):
    n, node_in = x.shape
    blk = 1000
    grid = n // blk
    w1x = W1[0:node_in]
    w1a = W1[node_in:node_in + 12]
    w1u = W1[node_in + 12:]
    batch3 = batch.astype(jnp.int32).reshape(grid, 1, blk)

    def plane(f, off):
        return f[off:off + n].reshape(grid, 1, blk)

    splanes = [plane(sums_f, (c * 4 + j) * NPAD)
               for c in range(NSC) for j in range(4)]
    cplanes = [plane(cnts_f, c * NPAD) for c in range(NSC)]
    mplanes = [plane(maxs_f, (c * 4 + j) * NPAD)
               for c in range(NSC) for j in range(4)]

    full = lambda shape: pl.BlockSpec(shape, lambda i: (0,) * len(shape))
    p3 = pl.BlockSpec((1, 1, blk), lambda i: (i, 0, 0))
    return pl.pallas_call(
        _mlp_body,
        grid=(grid,),
        in_specs=[
            pl.BlockSpec((blk, node_in), lambda i: (i, 0)),
            p3,
            full(u.shape),
            full(w1x.shape),
            full(w1a.shape),
            full(w1u.shape),
            full((1, 128)),
            full(W2.shape),
            full((1, 128)),
        ] + [p3] * 18,
        out_specs=pl.BlockSpec((blk, 128), lambda i: (i, 0)),
        out_shape=jax.ShapeDtypeStruct((n, 128), jnp.float32),
        compiler_params=pltpu.CompilerParams(
            dimension_semantics=("arbitrary",),
        ),
    )(x, batch3, u, w1x, w1a, w1u, b1.reshape(1, 128), W2,
      b2.reshape(1, 128), *splanes, *cplanes, *mplanes)


def kernel(x, edge_index, edge_attr, u, batch, W1, b1, W2, b2):
    n = x.shape[0]
    eiflat = edge_index.astype(jnp.int32).reshape(-1)
    aflat = edge_attr.T.reshape(-1)
    eiflat, aflat = lax.optimization_barrier((eiflat, aflat))
    sums_f, cnts_f, maxs_f = _run_sc_agg(eiflat, aflat)
    return _run_mlp(x, sums_f, cnts_f, maxs_f, batch, u, W1, b1, W2, b2)


# speedup vs baseline: 1.5647x; 1.1838x over previous
import functools

import jax
import jax.numpy as jnp
from jax import lax
from jax.experimental import pallas as pl
from jax.experimental.pallas import tpu as pltpu
from jax.experimental.pallas import tpu_sc as plsc


N_NODES = 100000
N_EDGES = 3200000
NSC = 2
NSUB = 16
RNG = 6256
NPAD = RNG * NSUB
RNG2 = RNG * 2
EPC = N_EDGES // NSC
W = 2000
NW = EPC // W
VPW = W // 16


def _sc_agg_body(eiflat_hbm, aflat_hbm, z1_hbm, negf_hbm, ones_hbm,
                 sums_out, cnts_out, maxs_out,
                 col0, col1, fattr0, fattr1, onesbuf, zbuf, stg, acc,
                 sp0, sp1, sp2, sp3, spc,
                 sem_c0, sem_c1, sem_a0, sem_a1, sem_add):
    c = lax.axis_index("c")
    s = lax.axis_index("s")
    lo = s * RNG
    ebase = c * EPC
    splanes = (sp0, sp1, sp2, sp3)
    mlo = lax.rem(s, 8) * RNG2
    pbase = lax.div(s, 8) * 2

    pltpu.sync_copy(z1_hbm, zbuf)
    for sp in (sp0, sp1, sp2, sp3, spc):
        for i in range(RNG // W):
            pltpu.sync_copy(zbuf, sp.at[pl.ds(lo + i * W, W)])
        rem = RNG % W
        pltpu.sync_copy(zbuf.at[pl.ds(0, rem)],
                        sp.at[pl.ds(lo + (RNG // W) * W, rem)])
    pltpu.sync_copy(negf_hbm, acc)
    pltpu.sync_copy(ones_hbm, onesbuf)
    plsc.subcore_barrier()

    iota16 = lax.iota(jnp.int32, 16)

    cbufs = (col0, col1)
    abufs = (fattr0, fattr1)
    csems = (sem_c0, sem_c1)
    asems = (sem_a0, sem_a1)

    def start(w, b):
        off = ebase + w * W
        pltpu.async_copy(eiflat_hbm.at[pl.ds(N_EDGES + off, W)], cbufs[b],
                         csems[b])
        for j in range(4):
            pltpu.async_copy(aflat_hbm.at[pl.ds(j * N_EDGES + off, W)],
                             abufs[b].at[pl.ds(j * W, W)], asems[b])

    def wait(w, b):
        off = ebase + w * W
        pltpu.make_async_copy(eiflat_hbm.at[pl.ds(N_EDGES + off, W)],
                              cbufs[b], csems[b]).wait()
        for j in range(4):
            pltpu.make_async_copy(aflat_hbm.at[pl.ds(j * N_EDGES + off, W)],
                                  abufs[b].at[pl.ds(j * W, W)],
                                  asems[b]).wait()

    start(0, 0)
    start(1, 1)

    def window_body(i, _):
        for b in (0, 1):
            w = 2 * i + b
            colb = cbufs[b]
            attrb = abufs[b]
            wait(w, b)

            mine = lax.rem(w, NSUB) == s

            @pl.when(mine)
            def _():
                for j in range(4):
                    pltpu.async_copy(attrb.at[pl.ds(j * W, W)],
                                     splanes[j].at[colb], sem_add, add=True)
                pltpu.async_copy(onesbuf, spc.at[colb], sem_add, add=True)

            def vreg_body(k, _):
                colv = colb[pl.ds(k * 16, 16)]
                loc = colv - mlo
                m = (loc >= 0) & (loc < RNG2)
                locs = jnp.where(m, loc, 0)
                avs = [attrb[pl.ds((pbase + jj) * W + k * 16, 16)]
                       for jj in range(2)]
                trash = 2 * RNG2 + iota16

                def rmw(mm):
                    bad = jnp.zeros((16,), jnp.bool_)
                    for jj in range(2):
                        idxj = jnp.where(mm, jj * RNG2 + locs, trash)
                        cur = plsc.load_gather(acc, [idxj])
                        new = jnp.maximum(cur, avs[jj])
                        plsc.store_scatter(acc, [idxj], new)
                        cur2 = plsc.load_gather(acc, [idxj])
                        bad = bad | (mm & (cur2 < avs[jj]))
                    return bad

                bad = rmw(m)
                lax.while_loop(lambda mm: jnp.any(mm), rmw, bad)
                return 0

            lax.fori_loop(0, VPW, vreg_body, 0)

            @pl.when(mine)
            def _():
                for j in range(4):
                    pltpu.make_async_copy(attrb.at[pl.ds(j * W, W)],
                                          splanes[j].at[colb],
                                          sem_add).wait()
                pltpu.make_async_copy(onesbuf, spc.at[colb], sem_add).wait()

            @pl.when(w + 2 < NW)
            def _():
                start(w + 2, b)
        return 0

    lax.fori_loop(0, NW // 2, window_body, 0)
    plsc.subcore_barrier()

    for j in range(4):
        pltpu.sync_copy(splanes[j].at[pl.ds(lo, RNG)], stg)
        pltpu.sync_copy(stg, sums_out.at[pl.ds((c * 4 + j) * NPAD + lo, RNG)])
    pltpu.sync_copy(spc.at[pl.ds(lo, RNG)], stg)
    pltpu.sync_copy(stg, cnts_out.at[pl.ds(c * NPAD + lo, RNG)])
    for jj in range(2):
        pltpu.sync_copy(
            acc.at[pl.ds(jj * RNG2, RNG2)],
            maxs_out.at[pl.ds((c * 4 + pbase + jj) * NPAD + mlo, RNG2)])


def _run_sc_agg(eiflat, aflat):
    mesh = plsc.VectorSubcoreMesh(core_axis_name="c", subcore_axis_name="s")
    z1 = jnp.zeros((W,), jnp.float32)
    negf = jnp.full((2 * RNG2 + 16,), -jnp.inf, jnp.float32)
    ones = jnp.ones((W,), jnp.float32)
    kfn = pl.kernel(
        _sc_agg_body,
        out_type=(
            jax.ShapeDtypeStruct((NSC * 4 * NPAD,), jnp.float32),
            jax.ShapeDtypeStruct((NSC * NPAD,), jnp.float32),
            jax.ShapeDtypeStruct((NSC * 4 * NPAD,), jnp.float32),
        ),
        mesh=mesh,
        scratch_types=[
            pltpu.VMEM((W,), jnp.int32),
            pltpu.VMEM((W,), jnp.int32),
            pltpu.VMEM((4 * W,), jnp.float32),
            pltpu.VMEM((4 * W,), jnp.float32),
            pltpu.VMEM((W,), jnp.float32),
            pltpu.VMEM((W,), jnp.float32),
            pltpu.VMEM((RNG,), jnp.float32),
            pltpu.VMEM((2 * RNG2 + 16,), jnp.float32),
            pltpu.VMEM_SHARED((NPAD,), jnp.float32),
            pltpu.VMEM_SHARED((NPAD,), jnp.float32),
            pltpu.VMEM_SHARED((NPAD,), jnp.float32),
            pltpu.VMEM_SHARED((NPAD,), jnp.float32),
            pltpu.VMEM_SHARED((NPAD,), jnp.float32),
            pltpu.SemaphoreType.DMA,
            pltpu.SemaphoreType.DMA,
            pltpu.SemaphoreType.DMA,
            pltpu.SemaphoreType.DMA,
            pltpu.SemaphoreType.DMA,
        ],
        compiler_params=pltpu.CompilerParams(needs_layout_passes=False,
                                             use_tc_tiling_on_sc=False),
    )
    return kfn(eiflat, aflat, z1, negf, ones)


def _mlp_body(x_ref, batch_ref, u_ref, w1x_ref, w1a_ref, w1u_ref, b1_ref,
              w2_ref, b2_ref, *rest):
    (s00, s01, s02, s03, s10, s11, s12, s13, c0, c1,
     m00, m01, m02, m03, m10, m11, m12, m13, out_ref) = rest
    s0p = (s00, s01, s02, s03)
    s1p = (s10, s11, s12, s13)
    m0p = (m00, m01, m02, m03)
    m1p = (m10, m11, m12, m13)

    cnt = (c0[0, 0, :] + c1[0, 0, :])[:, None]
    sj = [(s0p[j][0, 0, :] + s1p[j][0, 0, :])[:, None] for j in range(4)]
    mj = [jnp.maximum(m0p[j][0, 0, :], m1p[j][0, 0, :])[:, None]
          for j in range(4)]
    pos = cnt > 0
    inv = 1.0 / jnp.maximum(cnt, 1.0)
    agg = jnp.concatenate(
        sj + [jnp.where(pos, m, 0.0) for m in mj] + [s * inv for s in sj],
        axis=1)

    b = batch_ref[0, 0, :]
    oneh = (b[:, None] == lax.broadcasted_iota(jnp.int32, (1, 16), 1)
            ).astype(jnp.float32)
    uw = jnp.dot(u_ref[...], w1u_ref[...], preferred_element_type=jnp.float32)

    h = jnp.dot(x_ref[...], w1x_ref[...], preferred_element_type=jnp.float32)
    h += jnp.dot(agg, w1a_ref[...], preferred_element_type=jnp.float32)
    h += jnp.dot(oneh, uw, preferred_element_type=jnp.float32)
    h = jnp.maximum(h + b1_ref[...], 0.0)
    out_ref[...] = jnp.dot(h, w2_ref[...],
                           preferred_element_type=jnp.float32) + b2_ref[...]


def _run_mlp(x, sums_f, cnts_f, maxs_f, batch, u, W1, b1, W2, b2):
    n, node_in = x.shape
    blk = 1000
    grid = n // blk
    w1x = W1[0:node_in]
    w1a = W1[node_in:node_in + 12]
    w1u = W1[node_in + 12:]
    batch3 = batch.astype(jnp.int32).reshape(grid, 1, blk)

    def plane(f, off):
        return f[off:off + n].reshape(grid, 1, blk)

    splanes = [plane(sums_f, (c * 4 + j) * NPAD)
               for c in range(NSC) for j in range(4)]
    cplanes = [plane(cnts_f, c * NPAD) for c in range(NSC)]
    mplanes = [plane(maxs_f, (c * 4 + j) * NPAD)
               for c in range(NSC) for j in range(4)]

    full = lambda shape: pl.BlockSpec(shape, lambda i: (0,) * len(shape))
    p3 = pl.BlockSpec((1, 1, blk), lambda i: (i, 0, 0))
    return pl.pallas_call(
        _mlp_body,
        grid=(grid,),
        in_specs=[
            pl.BlockSpec((blk, node_in), lambda i: (i, 0)),
            p3,
            full(u.shape),
            full(w1x.shape),
            full(w1a.shape),
            full(w1u.shape),
            full((1, 128)),
            full(W2.shape),
            full((1, 128)),
        ] + [p3] * 18,
        out_specs=pl.BlockSpec((blk, 128), lambda i: (i, 0)),
        out_shape=jax.ShapeDtypeStruct((n, 128), jnp.float32),
        compiler_params=pltpu.CompilerParams(
            dimension_semantics=("arbitrary",),
        ),
    )(x, batch3, u, w1x, w1a, w1u, b1.reshape(1, 128), W2,
      b2.reshape(1, 128), *splanes, *cplanes, *mplanes)


def kernel(x, edge_index, edge_attr, u, batch, W1, b1, W2, b2):
    n = x.shape[0]
    eiflat = edge_index.astype(jnp.int32).reshape(-1)
    aflat = edge_attr.T.reshape(-1)
    eiflat, aflat = lax.optimization_barrier((eiflat, aflat))
    sums_f, cnts_f, maxs_f = _run_sc_agg(eiflat, aflat)
    return _run_mlp(x, sums_f, cnts_f, maxs_f, batch, u, W1, b1, W2, b2)


# speedup vs baseline: 1.7583x; 1.1237x over previous
import functools

import jax
import jax.numpy as jnp
from jax import lax
from jax.experimental import pallas as pl
from jax.experimental.pallas import tpu as pltpu
from jax.experimental.pallas import tpu_sc as plsc


N_NODES = 100000
N_EDGES = 3200000
NSC = 2
NSUB = 16
RNG = 6256
NPAD = RNG * NSUB
RNG2 = RNG * 4
EPC = N_EDGES // NSC
W = 2000
NW = EPC // W
VPW = W // 16


def _sc_agg_body(eiflat_hbm, aflat_hbm, z1_hbm, negf_hbm, ones_hbm,
                 sums_out, cnts_out, maxs_out,
                 col0, col1, fattr0, fattr1, onesbuf, zbuf, stg, acc,
                 sp0, sp1, sp2, sp3, spc,
                 sem_c0, sem_c1, sem_a0, sem_a1, sem_add):
    c = lax.axis_index("c")
    s = lax.axis_index("s")
    lo = s * RNG
    ebase = c * EPC
    splanes = (sp0, sp1, sp2, sp3)
    mlo = lax.rem(s, 4) * RNG2
    pbase = lax.div(s, 4)

    pltpu.sync_copy(z1_hbm, zbuf)
    for sp in (sp0, sp1, sp2, sp3, spc):
        for i in range(RNG // W):
            pltpu.sync_copy(zbuf, sp.at[pl.ds(lo + i * W, W)])
        rem = RNG % W
        pltpu.sync_copy(zbuf.at[pl.ds(0, rem)],
                        sp.at[pl.ds(lo + (RNG // W) * W, rem)])
    pltpu.sync_copy(negf_hbm, acc)
    pltpu.sync_copy(ones_hbm, onesbuf)
    plsc.subcore_barrier()

    iota16 = lax.iota(jnp.int32, 16)

    cbufs = (col0, col1)
    abufs = (fattr0, fattr1)
    csems = (sem_c0, sem_c1)
    asems = (sem_a0, sem_a1)

    def start(w, b):
        off = ebase + w * W
        pltpu.async_copy(eiflat_hbm.at[pl.ds(N_EDGES + off, W)], cbufs[b],
                         csems[b])
        for j in range(4):
            pltpu.async_copy(aflat_hbm.at[pl.ds(j * N_EDGES + off, W)],
                             abufs[b].at[pl.ds(j * W, W)], asems[b])

    def wait(w, b):
        off = ebase + w * W
        pltpu.make_async_copy(eiflat_hbm.at[pl.ds(N_EDGES + off, W)],
                              cbufs[b], csems[b]).wait()
        for j in range(4):
            pltpu.make_async_copy(aflat_hbm.at[pl.ds(j * N_EDGES + off, W)],
                                  abufs[b].at[pl.ds(j * W, W)],
                                  asems[b]).wait()

    start(0, 0)
    start(1, 1)

    def window_body(i, _):
        for b in (0, 1):
            w = 2 * i + b
            colb = cbufs[b]
            attrb = abufs[b]
            wait(w, b)

            mine = lax.rem(w, NSUB) == s

            @pl.when(mine)
            def _():
                for j in range(4):
                    pltpu.async_copy(attrb.at[pl.ds(j * W, W)],
                                     splanes[j].at[colb], sem_add, add=True)
                pltpu.async_copy(onesbuf, spc.at[colb], sem_add, add=True)

            def vreg_body(k, _):
                colv = colb[pl.ds(k * 16, 16)]
                loc = colv - mlo
                m = (loc >= 0) & (loc < RNG2)
                locs = jnp.where(m, loc, 0)
                av = attrb[pl.ds(pbase * W + k * 16, 16)]
                trash = RNG2 + iota16

                def rmw(mm):
                    idxj = jnp.where(mm, locs, trash)
                    cur = plsc.load_gather(acc, [idxj])
                    new = jnp.maximum(cur, av)
                    plsc.store_scatter(acc, [idxj], new)
                    cur2 = plsc.load_gather(acc, [idxj])
                    return mm & (cur2 < av)

                bad = rmw(m)
                lax.while_loop(lambda mm: jnp.any(mm), rmw, bad)
                return 0

            lax.fori_loop(0, VPW, vreg_body, 0)

            @pl.when(mine)
            def _():
                for j in range(4):
                    pltpu.make_async_copy(attrb.at[pl.ds(j * W, W)],
                                          splanes[j].at[colb],
                                          sem_add).wait()
                pltpu.make_async_copy(onesbuf, spc.at[colb], sem_add).wait()

            @pl.when(w + 2 < NW)
            def _():
                start(w + 2, b)
        return 0

    lax.fori_loop(0, NW // 2, window_body, 0)
    plsc.subcore_barrier()

    for j in range(4):
        pltpu.sync_copy(splanes[j].at[pl.ds(lo, RNG)], stg)
        pltpu.sync_copy(stg, sums_out.at[pl.ds((c * 4 + j) * NPAD + lo, RNG)])
    pltpu.sync_copy(spc.at[pl.ds(lo, RNG)], stg)
    pltpu.sync_copy(stg, cnts_out.at[pl.ds(c * NPAD + lo, RNG)])
    pltpu.sync_copy(
        acc.at[pl.ds(0, RNG2)],
        maxs_out.at[pl.ds((c * 4 + pbase) * NPAD + mlo, RNG2)])


def _run_sc_agg(eiflat, aflat):
    mesh = plsc.VectorSubcoreMesh(core_axis_name="c", subcore_axis_name="s")
    z1 = jnp.zeros((W,), jnp.float32)
    negf = jnp.full((RNG2 + 16,), -jnp.inf, jnp.float32)
    ones = jnp.ones((W,), jnp.float32)
    kfn = pl.kernel(
        _sc_agg_body,
        out_type=(
            jax.ShapeDtypeStruct((NSC * 4 * NPAD,), jnp.float32),
            jax.ShapeDtypeStruct((NSC * NPAD,), jnp.float32),
            jax.ShapeDtypeStruct((NSC * 4 * NPAD,), jnp.float32),
        ),
        mesh=mesh,
        scratch_types=[
            pltpu.VMEM((W,), jnp.int32),
            pltpu.VMEM((W,), jnp.int32),
            pltpu.VMEM((4 * W,), jnp.float32),
            pltpu.VMEM((4 * W,), jnp.float32),
            pltpu.VMEM((W,), jnp.float32),
            pltpu.VMEM((W,), jnp.float32),
            pltpu.VMEM((RNG,), jnp.float32),
            pltpu.VMEM((RNG2 + 16,), jnp.float32),
            pltpu.VMEM_SHARED((NPAD,), jnp.float32),
            pltpu.VMEM_SHARED((NPAD,), jnp.float32),
            pltpu.VMEM_SHARED((NPAD,), jnp.float32),
            pltpu.VMEM_SHARED((NPAD,), jnp.float32),
            pltpu.VMEM_SHARED((NPAD,), jnp.float32),
            pltpu.SemaphoreType.DMA,
            pltpu.SemaphoreType.DMA,
            pltpu.SemaphoreType.DMA,
            pltpu.SemaphoreType.DMA,
            pltpu.SemaphoreType.DMA,
        ],
        compiler_params=pltpu.CompilerParams(needs_layout_passes=False,
                                             use_tc_tiling_on_sc=False),
    )
    return kfn(eiflat, aflat, z1, negf, ones)


def _mlp_body(x_ref, batch_ref, u_ref, w1x_ref, w1a_ref, w1u_ref, b1_ref,
              w2_ref, b2_ref, *rest):
    (s00, s01, s02, s03, s10, s11, s12, s13, c0, c1,
     m00, m01, m02, m03, m10, m11, m12, m13, out_ref) = rest
    s0p = (s00, s01, s02, s03)
    s1p = (s10, s11, s12, s13)
    m0p = (m00, m01, m02, m03)
    m1p = (m10, m11, m12, m13)

    cnt = (c0[0, 0, :] + c1[0, 0, :])[:, None]
    sj = [(s0p[j][0, 0, :] + s1p[j][0, 0, :])[:, None] for j in range(4)]
    mj = [jnp.maximum(m0p[j][0, 0, :], m1p[j][0, 0, :])[:, None]
          for j in range(4)]
    pos = cnt > 0
    inv = 1.0 / jnp.maximum(cnt, 1.0)
    agg = jnp.concatenate(
        sj + [jnp.where(pos, m, 0.0) for m in mj] + [s * inv for s in sj],
        axis=1)

    b = batch_ref[0, 0, :]
    oneh = (b[:, None] == lax.broadcasted_iota(jnp.int32, (1, 16), 1)
            ).astype(jnp.float32)
    uw = jnp.dot(u_ref[...], w1u_ref[...], preferred_element_type=jnp.float32)

    h = jnp.dot(x_ref[...], w1x_ref[...], preferred_element_type=jnp.float32)
    h += jnp.dot(agg, w1a_ref[...], preferred_element_type=jnp.float32)
    h += jnp.dot(oneh, uw, preferred_element_type=jnp.float32)
    h = jnp.maximum(h + b1_ref[...], 0.0)
    out_ref[...] = jnp.dot(h, w2_ref[...],
                           preferred_element_type=jnp.float32) + b2_ref[...]


def _run_mlp(x, sums_f, cnts_f, maxs_f, batch, u, W1, b1, W2, b2):
    n, node_in = x.shape
    blk = 1000
    grid = n // blk
    w1x = W1[0:node_in]
    w1a = W1[node_in:node_in + 12]
    w1u = W1[node_in + 12:]
    batch3 = batch.astype(jnp.int32).reshape(grid, 1, blk)

    def plane(f, off):
        return f[off:off + n].reshape(grid, 1, blk)

    splanes = [plane(sums_f, (c * 4 + j) * NPAD)
               for c in range(NSC) for j in range(4)]
    cplanes = [plane(cnts_f, c * NPAD) for c in range(NSC)]
    mplanes = [plane(maxs_f, (c * 4 + j) * NPAD)
               for c in range(NSC) for j in range(4)]

    full = lambda shape: pl.BlockSpec(shape, lambda i: (0,) * len(shape))
    p3 = pl.BlockSpec((1, 1, blk), lambda i: (i, 0, 0))
    return pl.pallas_call(
        _mlp_body,
        grid=(grid,),
        in_specs=[
            pl.BlockSpec((blk, node_in), lambda i: (i, 0)),
            p3,
            full(u.shape),
            full(w1x.shape),
            full(w1a.shape),
            full(w1u.shape),
            full((1, 128)),
            full(W2.shape),
            full((1, 128)),
        ] + [p3] * 18,
        out_specs=pl.BlockSpec((blk, 128), lambda i: (i, 0)),
        out_shape=jax.ShapeDtypeStruct((n, 128), jnp.float32),
        compiler_params=pltpu.CompilerParams(
            dimension_semantics=("arbitrary",),
        ),
    )(x, batch3, u, w1x, w1a, w1u, b1.reshape(1, 128), W2,
      b2.reshape(1, 128), *splanes, *cplanes, *mplanes)


def kernel(x, edge_index, edge_attr, u, batch, W1, b1, W2, b2):
    n = x.shape[0]
    eiflat = edge_index.astype(jnp.int32).reshape(-1)
    aflat = edge_attr.T.reshape(-1)
    eiflat, aflat = lax.optimization_barrier((eiflat, aflat))
    sums_f, cnts_f, maxs_f = _run_sc_agg(eiflat, aflat)
    return _run_mlp(x, sums_f, cnts_f, maxs_f, batch, u, W1, b1, W2, b2)
